# Initial kernel scaffold; baseline (speedup 1.0000x reference)
import jax
import jax.numpy as jnp
from jax.experimental import pallas as pl


def kernel(ag_x, ag_edge_index, ag_x_batch, ab_x, ab_edge_index, ab_x_batch, W1_ag, b1_ag, W2_ag, b2_ag, W1_ab, b1_ab, W2_ab, b2_ab, g1_ag, be1_ag, g2_ag, be2_ag, g1_ab, be1_ab, g2_ab, be2_ab, Wg1, as1, ad1, bg1, Wg2, as2, ad2, bg2, Wc1_ag, bc1_ag, Wc2_ag, bc2_ag, Wc1_ab, bc1_ab, Wc2_ab, bc2_ab):
    raise NotImplementedError("write your pallas kernel here")


# speedup vs baseline: 1.3340x; 1.3340x over previous
import jax
import jax.numpy as jnp
from jax import lax
from jax.experimental import pallas as pl
from jax.experimental.pallas import tpu as pltpu

f32 = jnp.float32
NBATCH = 128
_NEG = -1e30


def _lrelu(x):
    return jnp.where(x >= 0, x, 0.2 * x)


def _dis_body(col_ref, dis_ref, acc_ref):
    i = pl.program_id(0)
    n = acc_ref.shape[0]
    ec = col_ref.shape[2]

    @pl.when(i == 0)
    def _():
        acc_ref[...] = jnp.zeros_like(acc_ref)

    col = col_ref[0]
    iota = lax.broadcasted_iota(jnp.int32, (n, ec), 0)
    onehot = (col == iota).astype(f32)
    acc_ref[...] += jnp.sum(onehot, axis=1, keepdims=True)

    @pl.when(i == pl.num_programs(0) - 1)
    def _():
        deg = acc_ref[...]
        dis_ref[...] = jnp.where(deg > 0, lax.rsqrt(jnp.maximum(deg, 1e-12)), 0.0)


def _dis(col3d, n):
    chunks, _, ec = col3d.shape
    return pl.pallas_call(
        _dis_body,
        grid=(chunks,),
        in_specs=[pl.BlockSpec((1, 1, ec), lambda i: (i, 0, 0))],
        out_specs=pl.BlockSpec((n, 1), lambda i: (0, 0)),
        out_shape=jax.ShapeDtypeStruct((n, 1), f32),
        scratch_shapes=[pltpu.VMEM((n, 1), f32)],
    )(col3d)


def _gcn_body(x_ref, w_ref, b_ref, g_ref, be_ref, dis_ref, row_ref, col_ref,
              out_ref, xw_ref, acc_ref):
    i = pl.program_id(0)
    n = xw_ref.shape[0]
    ec = row_ref.shape[2]

    @pl.when(i == 0)
    def _():
        xw = jnp.dot(x_ref[...], w_ref[...], preferred_element_type=f32)
        xw_ref[...] = xw * dis_ref[...]
        acc_ref[...] = jnp.zeros_like(acc_ref)

    iota = lax.broadcasted_iota(jnp.int32, (n, ec), 0)
    onehot_r = (row_ref[0] == iota).astype(f32)
    onehot_c = (col_ref[0] == iota).astype(f32)
    vals = lax.dot_general(onehot_r, xw_ref[...], (((0,), (0,)), ((), ())),
                           preferred_element_type=f32)
    acc_ref[...] += lax.dot_general(onehot_c, vals, (((1,), (0,)), ((), ())),
                                    preferred_element_type=f32)

    @pl.when(i == pl.num_programs(0) - 1)
    def _():
        h = jnp.tanh(acc_ref[...] * dis_ref[...] + b_ref[...])
        m = jnp.mean(h, axis=0, keepdims=True)
        v = jnp.mean((h - m) ** 2, axis=0, keepdims=True)
        out_ref[...] = (h - m) * lax.rsqrt(v + 1e-5) * g_ref[...] + be_ref[...]


def _gcn(x, w, b, g, be, dis, row3d, col3d):
    n, din = x.shape
    hid = w.shape[1]
    chunks, _, ec = row3d.shape
    return pl.pallas_call(
        _gcn_body,
        grid=(chunks,),
        in_specs=[
            pl.BlockSpec((n, din), lambda i: (0, 0)),
            pl.BlockSpec((din, hid), lambda i: (0, 0)),
            pl.BlockSpec((1, hid), lambda i: (0, 0)),
            pl.BlockSpec((1, hid), lambda i: (0, 0)),
            pl.BlockSpec((1, hid), lambda i: (0, 0)),
            pl.BlockSpec((n, 1), lambda i: (0, 0)),
            pl.BlockSpec((1, 1, ec), lambda i: (i, 0, 0)),
            pl.BlockSpec((1, 1, ec), lambda i: (i, 0, 0)),
        ],
        out_specs=pl.BlockSpec((n, hid), lambda i: (0, 0)),
        out_shape=jax.ShapeDtypeStruct((n, hid), f32),
        scratch_shapes=[pltpu.VMEM((n, hid), f32), pltpu.VMEM((n, hid), f32)],
    )(x, w, b, g, be, dis, row3d, col3d)


def _gat_body(hag_f, hag_t, hab_t, bag_col, bab_col_t, bab_row_t, w_ref,
              as_ref, ad_ref, bias_ref, oag_ref, oab_ref,
              xlag_ref, asn_ref, m_ref, eye_ref, *, heads, c, concat):
    i = pl.program_id(0)
    n = hag_f.shape[0]
    kt = hab_t.shape[0]

    @pl.when(i == 0)
    def _():
        xlag = jnp.dot(hag_f[...], w_ref[...], preferred_element_type=f32)
        xlag_ref[...] = xlag
        iotab = lax.broadcasted_iota(jnp.int32, (n, NBATCH), 1)
        onehot_bag = bag_col[...] == iotab
        ii = lax.broadcasted_iota(jnp.int32, (kt, kt), 0)
        jj = lax.broadcasted_iota(jnp.int32, (kt, kt), 1)
        eye_ref[...] = (ii == jj).astype(f32)
        for h in range(heads):
            xl_h = xlag[:, h * c:(h + 1) * c]
            asn_h = jnp.sum(xl_h * as_ref[h:h + 1, :], axis=1, keepdims=True)
            asn_ref[:, h:h + 1] = asn_h
            m_ref[h:h + 1, :] = jnp.max(
                jnp.where(onehot_bag, asn_h, _NEG), axis=0, keepdims=True)

    eye = eye_ref[...]

    def tr(colv):
        return lax.dot_general(colv, eye, (((0,), (0,)), ((), ())),
                               preferred_element_type=f32)

    def trr(rowv):
        return lax.dot_general(eye, rowv, (((1,), (1,)), ((), ())),
                               preferred_element_type=f32)

    xlag_t = jnp.dot(hag_t[...], w_ref[...], preferred_element_type=f32)
    xlab_t = jnp.dot(hab_t[...], w_ref[...], preferred_element_type=f32)
    mask = bag_col[...] == bab_row_t[...]
    iotab_t = lax.broadcasted_iota(jnp.int32, (kt, NBATCH), 1)
    onehot_bab = bab_col_t[...] == iotab_t
    ag_parts = []
    ab_parts = []
    for h in range(heads):
        xlag_h = xlag_ref[:, h * c:(h + 1) * c]
        xlab_h = xlab_t[:, h * c:(h + 1) * c]
        asn_h = asn_ref[:, h:h + 1]
        adn_col = jnp.sum(xlab_h * ad_ref[h:h + 1, :], axis=1, keepdims=True)
        asnab_col = jnp.sum(xlab_h * as_ref[h:h + 1, :], axis=1, keepdims=True)
        mg_col = jnp.max(jnp.where(onehot_bab, m_ref[h:h + 1, :], _NEG),
                         axis=1, keepdims=True)
        al_self = _lrelu(asnab_col + adn_col)
        amax_col = jnp.maximum(_lrelu(mg_col + adn_col), al_self)
        aself_col = jnp.exp(al_self - amax_col)
        adn_row = tr(adn_col)
        amax_row = tr(amax_col)
        al = _lrelu(asn_h + adn_row)
        ae = jnp.where(mask, jnp.exp(al - amax_row), 0.0)
        den_col = trr(jnp.sum(ae, axis=0, keepdims=True))
        vsum = lax.dot_general(ae, xlag_h, (((0,), (0,)), ((), ())),
                               preferred_element_type=f32)
        num = vsum + aself_col * xlab_h
        ab_parts.append(num / (den_col + aself_col + 1e-16))
        ag_parts.append(xlag_t[:, h * c:(h + 1) * c])
    if concat:
        oag = jnp.concatenate(ag_parts, axis=1)
        oab = jnp.concatenate(ab_parts, axis=1)
    else:
        oag = sum(ag_parts) * (1.0 / heads)
        oab = sum(ab_parts) * (1.0 / heads)
    oag_ref[...] = jnp.tanh(oag + bias_ref[...])
    oab_ref[...] = jnp.tanh(oab + bias_ref[...])


def _gat(hag, hab, bag_col, bab_col, bab_row, w, a_s, a_d, bias, concat, kt):
    import functools
    n, din = hag.shape
    heads, c = a_s.shape
    out_d = heads * c if concat else c
    body = functools.partial(_gat_body, heads=heads, c=c, concat=concat)
    return pl.pallas_call(
        body,
        grid=(n // kt,),
        in_specs=[
            pl.BlockSpec((n, din), lambda i: (0, 0)),
            pl.BlockSpec((kt, din), lambda i: (i, 0)),
            pl.BlockSpec((kt, din), lambda i: (i, 0)),
            pl.BlockSpec((n, 1), lambda i: (0, 0)),
            pl.BlockSpec((kt, 1), lambda i: (i, 0)),
            pl.BlockSpec((1, kt), lambda i: (0, i)),
            pl.BlockSpec((din, heads * c), lambda i: (0, 0)),
            pl.BlockSpec((heads, c), lambda i: (0, 0)),
            pl.BlockSpec((heads, c), lambda i: (0, 0)),
            pl.BlockSpec((1, out_d), lambda i: (0, 0)),
        ],
        out_specs=[
            pl.BlockSpec((kt, out_d), lambda i: (i, 0)),
            pl.BlockSpec((kt, out_d), lambda i: (i, 0)),
        ],
        out_shape=[
            jax.ShapeDtypeStruct((n, out_d), f32),
            jax.ShapeDtypeStruct((n, out_d), f32),
        ],
        scratch_shapes=[
            pltpu.VMEM((n, heads * c), f32),
            pltpu.VMEM((n, heads), f32),
            pltpu.VMEM((8, NBATCH), f32),
            pltpu.VMEM((kt, kt), f32),
        ],
    )(hag, hag, hab, bag_col, bab_col, bab_row, w, a_s, a_d, bias)


def _mlp_body(h1_ref, h2_ref, w1_ref, b1_ref, w2_ref, b2_ref, out_ref):
    cat = jnp.concatenate([h1_ref[...], h2_ref[...]], axis=1)
    t = jnp.tanh(jnp.dot(cat, w1_ref[...], preferred_element_type=f32)
                 + b1_ref[...])
    out_ref[...] = jnp.dot(t, w2_ref[...], preferred_element_type=f32) + b2_ref[...]


def _mlp(h1, h2, w1, b1, w2, b2):
    n, d1 = h1.shape
    d2 = h2.shape[1]
    hid = w1.shape[1]
    return pl.pallas_call(
        _mlp_body,
        in_specs=[
            pl.BlockSpec((n, d1), lambda: (0, 0)),
            pl.BlockSpec((n, d2), lambda: (0, 0)),
            pl.BlockSpec((d1 + d2, hid), lambda: (0, 0)),
            pl.BlockSpec((1, hid), lambda: (0, 0)),
            pl.BlockSpec((hid, 1), lambda: (0, 0)),
            pl.BlockSpec((1, 1), lambda: (0, 0)),
        ],
        out_specs=pl.BlockSpec((n, 1), lambda: (0, 0)),
        out_shape=jax.ShapeDtypeStruct((n, 1), f32),
    )(h1, h2, w1, b1, w2, b2)


def _edges3d(ei, n, ec):
    sl = jnp.arange(n, dtype=ei.dtype)
    row = jnp.concatenate([ei[0], sl])
    col = jnp.concatenate([ei[1], sl])
    chunks = row.shape[0] // ec
    return row.reshape(chunks, 1, ec), col.reshape(chunks, 1, ec)


def kernel(ag_x, ag_edge_index, ag_x_batch, ab_x, ab_edge_index, ab_x_batch,
           W1_ag, b1_ag, W2_ag, b2_ag, W1_ab, b1_ab, W2_ab, b2_ab,
           g1_ag, be1_ag, g2_ag, be2_ag, g1_ab, be1_ab, g2_ab, be2_ab,
           Wg1, as1, ad1, bg1, Wg2, as2, ad2, bg2,
           Wc1_ag, bc1_ag, Wc2_ag, bc2_ag, Wc1_ab, bc1_ab, Wc2_ab, bc2_ab,
           *, ec=512, kt=256):
    n_ag = ag_x.shape[0]
    n_ab = ab_x.shape[0]
    r2 = lambda v: v.reshape(1, -1)

    rag3, cag3 = _edges3d(ag_edge_index, n_ag, ec)
    rab3, cab3 = _edges3d(ab_edge_index, n_ab, ec)
    dis_ag = _dis(cag3, n_ag)
    dis_ab = _dis(cab3, n_ab)

    ag_h1 = _gcn(ag_x, W1_ag, r2(b1_ag), r2(g1_ag), r2(be1_ag), dis_ag, rag3, cag3)
    ag_h1 = _gcn(ag_h1, W2_ag, r2(b2_ag), r2(g2_ag), r2(be2_ag), dis_ag, rag3, cag3)
    ab_h1 = _gcn(ab_x, W1_ab, r2(b1_ab), r2(g1_ab), r2(be1_ab), dis_ab, rab3, cab3)
    ab_h1 = _gcn(ab_h1, W2_ab, r2(b2_ab), r2(g2_ab), r2(be2_ab), dis_ab, rab3, cab3)

    bag_col = ag_x_batch.reshape(n_ag, 1)
    bab_col = ab_x_batch.reshape(n_ab, 1)
    bab_row = ab_x_batch.reshape(1, n_ab)
    ag_g1, ab_g1 = _gat(ag_h1, ab_h1, bag_col, bab_col, bab_row,
                        Wg1, as1, ad1, r2(bg1), True, kt)
    ag_h2, ab_h2 = _gat(ag_g1, ab_g1, bag_col, bab_col, bab_row,
                        Wg2, as2, ad2, r2(bg2), False, kt)

    ag_out = _mlp(ag_h1, ag_h2, Wc1_ag, r2(bc1_ag), Wc2_ag, r2(bc2_ag))
    ab_out = _mlp(ab_h1, ab_h2, Wc1_ab, r2(bc1_ab), Wc2_ab, r2(bc2_ab))
    return (ag_out, ag_h1, ag_h2, ab_out, ab_h1, ab_h2)


# speedup vs baseline: 6.8553x; 5.1388x over previous
import functools

import jax
import jax.numpy as jnp
from jax import lax
from jax.experimental import pallas as pl
from jax.experimental.pallas import tpu as pltpu
from jax.experimental.pallas import tpu_sc as plsc

f32 = jnp.float32
i32 = jnp.int32
NBATCH = 128
_NEG = -1e30
NW = 32
NSUB = 16
KCH = 128


def _lrelu(x):
    return jnp.where(x >= 0, x, 0.2 * x)


def _dis_of(degp_ref):
    deg = degp_ref[0] + degp_ref[1]
    return jnp.where(deg > 0, lax.rsqrt(jnp.maximum(deg, 1e-12)), 0.0)


def _sc_segsum(y, ridx, cidx, npad):
    d = y.shape[1]
    ch = ridx.shape[1]
    rows_per = npad // NSUB
    mesh = plsc.VectorSubcoreMesh(core_axis_name="c", subcore_axis_name="s")

    @functools.partial(
        pl.kernel, mesh=mesh,
        out_type=jax.ShapeDtypeStruct((2 * npad, d), f32),
        compiler_params=pltpu.CompilerParams(use_tc_tiling_on_sc=False),
        scratch_types=[
            pltpu.VMEM((ch, KCH), i32),
            pltpu.VMEM((ch, KCH), i32),
            pltpu.VMEM((KCH, d), f32),
            pltpu.VMEM_SHARED((npad, d), f32),
            pltpu.SemaphoreType.DMA,
        ],
    )
    def seg(y_hbm, r_hbm, c_hbm, z_hbm, out_hbm, ridx_v, cidx_v, rows_v,
            acc_sh, sem):
        c = lax.axis_index("c")
        s = lax.axis_index("s")
        w = c * NSUB + s
        pltpu.sync_copy(z_hbm.at[pl.ds(s * rows_per, rows_per)],
                        acc_sh.at[pl.ds(s * rows_per, rows_per)])
        pltpu.sync_copy(r_hbm.at[w], ridx_v)
        pltpu.sync_copy(c_hbm.at[w], cidx_v)
        plsc.subcore_barrier()
        for j in range(ch):
            pltpu.async_copy(y_hbm.at[ridx_v.at[j]], rows_v, sem).wait()
            pltpu.sync_copy(rows_v, acc_sh.at[cidx_v.at[j]], add=True)
        plsc.subcore_barrier()
        pltpu.sync_copy(acc_sh.at[pl.ds(s * rows_per, rows_per)],
                        out_hbm.at[pl.ds(c * npad + s * rows_per, rows_per)])

    return seg(y, ridx, cidx, jnp.zeros((npad, d), f32))


def _sc_deg(cidx, npad):
    d = 16
    ch = cidx.shape[1]
    rows_per = npad // NSUB
    mesh = plsc.VectorSubcoreMesh(core_axis_name="c", subcore_axis_name="s")

    @functools.partial(
        pl.kernel, mesh=mesh,
        out_type=jax.ShapeDtypeStruct((2 * npad, d), f32),
        compiler_params=pltpu.CompilerParams(use_tc_tiling_on_sc=False),
        scratch_types=[
            pltpu.VMEM((ch, KCH), i32),
            pltpu.VMEM((KCH, d), f32),
            pltpu.VMEM_SHARED((npad, d), f32),
        ],
    )
    def deg(c_hbm, ones_hbm, z_hbm, out_hbm, cidx_v, ones_v, acc_sh):
        c = lax.axis_index("c")
        s = lax.axis_index("s")
        w = c * NSUB + s
        pltpu.sync_copy(z_hbm.at[pl.ds(s * rows_per, rows_per)],
                        acc_sh.at[pl.ds(s * rows_per, rows_per)])
        pltpu.sync_copy(c_hbm.at[w], cidx_v)
        pltpu.sync_copy(ones_hbm, ones_v)
        plsc.subcore_barrier()
        for j in range(ch):
            pltpu.sync_copy(ones_v, acc_sh.at[cidx_v.at[j]], add=True)
        plsc.subcore_barrier()
        pltpu.sync_copy(acc_sh.at[pl.ds(s * rows_per, rows_per)],
                        out_hbm.at[pl.ds(c * npad + s * rows_per, rows_per)])

    return deg(cidx, jnp.ones((KCH, d), f32), jnp.zeros((npad, d), f32))


def _pre_body(x_ref, w_ref, degp_ref, y_ref):
    dis = _dis_of(degp_ref)
    y_ref[...] = jnp.dot(x_ref[...], w_ref[...],
                         preferred_element_type=f32) * dis


def _gcn_pre(x, w, degp):
    n, din = x.shape
    hid = w.shape[1]
    return pl.pallas_call(
        _pre_body,
        in_specs=[
            pl.BlockSpec((n, din), lambda: (0, 0)),
            pl.BlockSpec((din, hid), lambda: (0, 0)),
            pl.BlockSpec((2, n, 1), lambda: (0, 0, 0)),
        ],
        out_specs=pl.BlockSpec((n, hid), lambda: (0, 0)),
        out_shape=jax.ShapeDtypeStruct((n, hid), f32),
    )(x, w, degp)


def _bn_tanh(s1, dis, b_ref, g_ref, be_ref):
    h = jnp.tanh(s1 * dis + b_ref[...])
    m = jnp.mean(h, axis=0, keepdims=True)
    v = jnp.mean((h - m) ** 2, axis=0, keepdims=True)
    return (h - m) * lax.rsqrt(v + 1e-5) * g_ref[...] + be_ref[...]


def _mid_body(parts_ref, degp_ref, b_ref, g_ref, be_ref, w2_ref, y2_ref):
    dis = _dis_of(degp_ref)
    t1 = _bn_tanh(parts_ref[0] + parts_ref[1], dis, b_ref, g_ref, be_ref)
    y2_ref[...] = jnp.dot(t1, w2_ref[...], preferred_element_type=f32) * dis


def _gcn_mid(parts, degp, b, g, be, w2):
    _, n, hid = parts.shape
    h2 = w2.shape[1]
    return pl.pallas_call(
        _mid_body,
        in_specs=[
            pl.BlockSpec((2, n, hid), lambda: (0, 0, 0)),
            pl.BlockSpec((2, n, 1), lambda: (0, 0, 0)),
            pl.BlockSpec((1, hid), lambda: (0, 0)),
            pl.BlockSpec((1, hid), lambda: (0, 0)),
            pl.BlockSpec((1, hid), lambda: (0, 0)),
            pl.BlockSpec((hid, h2), lambda: (0, 0)),
        ],
        out_specs=pl.BlockSpec((n, h2), lambda: (0, 0)),
        out_shape=jax.ShapeDtypeStruct((n, h2), f32),
    )(parts, degp, b, g, be, w2)


def _post_body(parts_ref, degp_ref, b_ref, g_ref, be_ref, h_ref):
    dis = _dis_of(degp_ref)
    h_ref[...] = _bn_tanh(parts_ref[0] + parts_ref[1], dis, b_ref, g_ref,
                          be_ref)


def _gcn_post(parts, degp, b, g, be):
    _, n, hid = parts.shape
    return pl.pallas_call(
        _post_body,
        in_specs=[
            pl.BlockSpec((2, n, hid), lambda: (0, 0, 0)),
            pl.BlockSpec((2, n, 1), lambda: (0, 0, 0)),
            pl.BlockSpec((1, hid), lambda: (0, 0)),
            pl.BlockSpec((1, hid), lambda: (0, 0)),
            pl.BlockSpec((1, hid), lambda: (0, 0)),
        ],
        out_specs=pl.BlockSpec((n, hid), lambda: (0, 0)),
        out_shape=jax.ShapeDtypeStruct((n, hid), f32),
    )(parts, degp, b, g, be)


def _gat_body(hag_f, hag_t, hab_t, bag_col, bab_col_t, bab_row_t, w_ref,
              as_ref, ad_ref, bias_ref, oag_ref, oab_ref,
              xlag_ref, asn_ref, m_ref, eye_ref, *, heads, c, concat):
    i = pl.program_id(0)
    n = hag_f.shape[0]
    kt = hab_t.shape[0]

    @pl.when(i == 0)
    def _():
        xlag = jnp.dot(hag_f[...], w_ref[...], preferred_element_type=f32)
        xlag_ref[...] = xlag
        iotab = lax.broadcasted_iota(i32, (n, NBATCH), 1)
        onehot_bag = bag_col[...] == iotab
        ii = lax.broadcasted_iota(i32, (kt, kt), 0)
        jj = lax.broadcasted_iota(i32, (kt, kt), 1)
        eye_ref[...] = (ii == jj).astype(f32)
        for h in range(heads):
            xl_h = xlag[:, h * c:(h + 1) * c]
            asn_h = jnp.sum(xl_h * as_ref[h:h + 1, :], axis=1, keepdims=True)
            asn_ref[:, h:h + 1] = asn_h
            m_ref[h:h + 1, :] = jnp.max(
                jnp.where(onehot_bag, asn_h, _NEG), axis=0, keepdims=True)

    eye = eye_ref[...]

    def tr(colv):
        return lax.dot_general(colv, eye, (((0,), (0,)), ((), ())),
                               preferred_element_type=f32)

    def trr(rowv):
        return lax.dot_general(eye, rowv, (((1,), (1,)), ((), ())),
                               preferred_element_type=f32)

    xlag_t = jnp.dot(hag_t[...], w_ref[...], preferred_element_type=f32)
    xlab_t = jnp.dot(hab_t[...], w_ref[...], preferred_element_type=f32)
    mask = bag_col[...] == bab_row_t[...]
    iotab_t = lax.broadcasted_iota(i32, (kt, NBATCH), 1)
    onehot_bab = bab_col_t[...] == iotab_t
    ag_parts = []
    ab_parts = []
    for h in range(heads):
        xlag_h = xlag_ref[:, h * c:(h + 1) * c]
        xlab_h = xlab_t[:, h * c:(h + 1) * c]
        asn_h = asn_ref[:, h:h + 1]
        adn_col = jnp.sum(xlab_h * ad_ref[h:h + 1, :], axis=1, keepdims=True)
        asnab_col = jnp.sum(xlab_h * as_ref[h:h + 1, :], axis=1, keepdims=True)
        mg_col = jnp.max(jnp.where(onehot_bab, m_ref[h:h + 1, :], _NEG),
                         axis=1, keepdims=True)
        al_self = _lrelu(asnab_col + adn_col)
        amax_col = jnp.maximum(_lrelu(mg_col + adn_col), al_self)
        aself_col = jnp.exp(al_self - amax_col)
        adn_row = tr(adn_col)
        amax_row = tr(amax_col)
        al = _lrelu(asn_h + adn_row)
        ae = jnp.where(mask, jnp.exp(al - amax_row), 0.0)
        den_col = trr(jnp.sum(ae, axis=0, keepdims=True))
        vsum = lax.dot_general(ae, xlag_h, (((0,), (0,)), ((), ())),
                               preferred_element_type=f32)
        num = vsum + aself_col * xlab_h
        ab_parts.append(num / (den_col + aself_col + 1e-16))
        ag_parts.append(xlag_t[:, h * c:(h + 1) * c])
    if concat:
        oag = jnp.concatenate(ag_parts, axis=1)
        oab = jnp.concatenate(ab_parts, axis=1)
    else:
        oag = sum(ag_parts) * (1.0 / heads)
        oab = sum(ab_parts) * (1.0 / heads)
    oag_ref[...] = jnp.tanh(oag + bias_ref[...])
    oab_ref[...] = jnp.tanh(oab + bias_ref[...])


def _gat(hag, hab, bag_col, bab_col, bab_row, w, a_s, a_d, bias, concat, kt):
    n, din = hag.shape
    heads, c = a_s.shape
    out_d = heads * c if concat else c
    body = functools.partial(_gat_body, heads=heads, c=c, concat=concat)
    return pl.pallas_call(
        body,
        grid=(n // kt,),
        in_specs=[
            pl.BlockSpec((n, din), lambda i: (0, 0)),
            pl.BlockSpec((kt, din), lambda i: (i, 0)),
            pl.BlockSpec((kt, din), lambda i: (i, 0)),
            pl.BlockSpec((n, 1), lambda i: (0, 0)),
            pl.BlockSpec((kt, 1), lambda i: (i, 0)),
            pl.BlockSpec((1, kt), lambda i: (0, i)),
            pl.BlockSpec((din, heads * c), lambda i: (0, 0)),
            pl.BlockSpec((heads, c), lambda i: (0, 0)),
            pl.BlockSpec((heads, c), lambda i: (0, 0)),
            pl.BlockSpec((1, out_d), lambda i: (0, 0)),
        ],
        out_specs=[
            pl.BlockSpec((kt, out_d), lambda i: (i, 0)),
            pl.BlockSpec((kt, out_d), lambda i: (i, 0)),
        ],
        out_shape=[
            jax.ShapeDtypeStruct((n, out_d), f32),
            jax.ShapeDtypeStruct((n, out_d), f32),
        ],
        scratch_shapes=[
            pltpu.VMEM((n, heads * c), f32),
            pltpu.VMEM((n, heads), f32),
            pltpu.VMEM((8, NBATCH), f32),
            pltpu.VMEM((kt, kt), f32),
        ],
    )(hag, hag, hab, bag_col, bab_col, bab_row, w, a_s, a_d, bias)


def _mlp_body(h1_ref, h2_ref, w1_ref, b1_ref, w2_ref, b2_ref, out_ref):
    cat = jnp.concatenate([h1_ref[...], h2_ref[...]], axis=1)
    t = jnp.tanh(jnp.dot(cat, w1_ref[...], preferred_element_type=f32)
                 + b1_ref[...])
    out_ref[...] = jnp.dot(t, w2_ref[...],
                           preferred_element_type=f32) + b2_ref[...]


def _mlp(h1, h2, w1, b1, w2, b2):
    n, d1 = h1.shape
    d2 = h2.shape[1]
    hid = w1.shape[1]
    return pl.pallas_call(
        _mlp_body,
        in_specs=[
            pl.BlockSpec((n, d1), lambda: (0, 0)),
            pl.BlockSpec((n, d2), lambda: (0, 0)),
            pl.BlockSpec((d1 + d2, hid), lambda: (0, 0)),
            pl.BlockSpec((1, hid), lambda: (0, 0)),
            pl.BlockSpec((hid, 1), lambda: (0, 0)),
            pl.BlockSpec((1, 1), lambda: (0, 0)),
        ],
        out_specs=pl.BlockSpec((n, 1), lambda: (0, 0)),
        out_shape=jax.ShapeDtypeStruct((n, 1), f32),
    )(h1, h2, w1, b1, w2, b2)


def _edges_sc(ei, n):
    npad = n + 8 * NSUB
    sl = jnp.arange(n, dtype=ei.dtype)
    row = jnp.concatenate([ei[0], sl])
    col = jnp.concatenate([ei[1], sl])
    e = row.shape[0]
    epad = -(-e // (NW * KCH)) * (NW * KCH)
    pad = epad - e
    fill = jnp.full((pad,), npad - 1, dtype=ei.dtype)
    row = jnp.concatenate([row, fill]).reshape(NW, -1, KCH)
    col = jnp.concatenate([col, fill]).reshape(NW, -1, KCH)
    return row, col, npad


def _split_parts(out2, n, npad, d):
    return jnp.stack([out2[0:n, :d], out2[npad:npad + n, :d]])


def _gcn_graph(x, w1, b1, g1, be1, w2, b2, g2, be2, ei):
    n = x.shape[0]
    hid = w1.shape[1]
    ridx, cidx, npad = _edges_sc(ei, n)
    degp = _split_parts(_sc_deg(cidx, npad), n, npad, 1)
    y1 = _gcn_pre(x, w1, degp)
    y1p = jnp.concatenate([y1, jnp.zeros((npad - n, hid), f32)])
    s1 = _split_parts(_sc_segsum(y1p, ridx, cidx, npad), n, npad, hid)
    y2 = _gcn_mid(s1, degp, b1, g1, be1, w2)
    y2p = jnp.concatenate([y2, jnp.zeros((npad - n, w2.shape[1]), f32)])
    s2 = _split_parts(_sc_segsum(y2p, ridx, cidx, npad), n, npad, w2.shape[1])
    return _gcn_post(s2, degp, b2, g2, be2)


def kernel(ag_x, ag_edge_index, ag_x_batch, ab_x, ab_edge_index, ab_x_batch,
           W1_ag, b1_ag, W2_ag, b2_ag, W1_ab, b1_ab, W2_ab, b2_ab,
           g1_ag, be1_ag, g2_ag, be2_ag, g1_ab, be1_ab, g2_ab, be2_ab,
           Wg1, as1, ad1, bg1, Wg2, as2, ad2, bg2,
           Wc1_ag, bc1_ag, Wc2_ag, bc2_ag, Wc1_ab, bc1_ab, Wc2_ab, bc2_ab,
           *, kt=256):
    n_ag = ag_x.shape[0]
    n_ab = ab_x.shape[0]
    r2 = lambda v: v.reshape(1, -1)

    ag_h1 = _gcn_graph(ag_x, W1_ag, r2(b1_ag), r2(g1_ag), r2(be1_ag),
                       W2_ag, r2(b2_ag), r2(g2_ag), r2(be2_ag), ag_edge_index)
    ab_h1 = _gcn_graph(ab_x, W1_ab, r2(b1_ab), r2(g1_ab), r2(be1_ab),
                       W2_ab, r2(b2_ab), r2(g2_ab), r2(be2_ab), ab_edge_index)

    bag_col = ag_x_batch.reshape(n_ag, 1)
    bab_col = ab_x_batch.reshape(n_ab, 1)
    bab_row = ab_x_batch.reshape(1, n_ab)
    ag_g1, ab_g1 = _gat(ag_h1, ab_h1, bag_col, bab_col, bab_row,
                        Wg1, as1, ad1, r2(bg1), True, kt)
    ag_h2, ab_h2 = _gat(ag_g1, ab_g1, bag_col, bab_col, bab_row,
                        Wg2, as2, ad2, r2(bg2), False, kt)

    ag_out = _mlp(ag_h1, ag_h2, Wc1_ag, r2(bc1_ag), Wc2_ag, r2(bc2_ag))
    ab_out = _mlp(ab_h1, ab_h2, Wc1_ab, r2(bc1_ab), Wc2_ab, r2(bc2_ab))
    return (ag_out, ag_h1, ag_h2, ab_out, ab_h1, ab_h2)


# speedup vs baseline: 10.4621x; 1.5261x over previous
import functools

import jax
import jax.numpy as jnp
from jax import lax
from jax.experimental import pallas as pl
from jax.experimental.pallas import tpu as pltpu
from jax.experimental.pallas import tpu_sc as plsc

f32 = jnp.float32
i32 = jnp.int32
NBATCH = 128
_NEG = -1e30
NW = 32
NSUB = 16
KCH = 128


def _lrelu(x):
    return jnp.where(x >= 0, x, 0.2 * x)


def _dis_of(degp_ref):
    deg = degp_ref[0] + degp_ref[1]
    return jnp.where(deg > 0, lax.rsqrt(jnp.maximum(deg, 1e-12)), 0.0)


def _sc_segsum(y, ridx, cidx, npad):
    d = y.shape[1]
    ch = ridx.shape[1]
    rows_per = npad // NSUB
    mesh = plsc.VectorSubcoreMesh(core_axis_name="c", subcore_axis_name="s")

    @functools.partial(
        pl.kernel, mesh=mesh,
        out_type=jax.ShapeDtypeStruct((2 * npad, d), f32),
        compiler_params=pltpu.CompilerParams(use_tc_tiling_on_sc=False),
        scratch_types=[
            pltpu.VMEM((ch, KCH), i32),
            pltpu.VMEM((ch, KCH), i32),
            pltpu.VMEM((KCH, d), f32),
            pltpu.VMEM_SHARED((npad, d), f32),
            pltpu.SemaphoreType.DMA,
        ],
    )
    def seg(y_hbm, r_hbm, c_hbm, z_hbm, out_hbm, ridx_v, cidx_v, rows_v,
            acc_sh, sem):
        c = lax.axis_index("c")
        s = lax.axis_index("s")
        w = c * NSUB + s
        pltpu.sync_copy(z_hbm.at[pl.ds(s * rows_per, rows_per)],
                        acc_sh.at[pl.ds(s * rows_per, rows_per)])
        pltpu.sync_copy(r_hbm.at[w], ridx_v)
        pltpu.sync_copy(c_hbm.at[w], cidx_v)
        plsc.subcore_barrier()
        for j in range(ch):
            pltpu.async_copy(y_hbm.at[ridx_v.at[j]], rows_v, sem).wait()
            pltpu.sync_copy(rows_v, acc_sh.at[cidx_v.at[j]], add=True)
        plsc.subcore_barrier()
        pltpu.sync_copy(acc_sh.at[pl.ds(s * rows_per, rows_per)],
                        out_hbm.at[pl.ds(c * npad + s * rows_per, rows_per)])

    return seg(y, ridx, cidx, jnp.zeros((npad, d), f32))


def _sc_deg(cidx, npad):
    d = 16
    ch = cidx.shape[1]
    rows_per = npad // NSUB
    mesh = plsc.VectorSubcoreMesh(core_axis_name="c", subcore_axis_name="s")

    @functools.partial(
        pl.kernel, mesh=mesh,
        out_type=jax.ShapeDtypeStruct((2 * npad, d), f32),
        compiler_params=pltpu.CompilerParams(use_tc_tiling_on_sc=False),
        scratch_types=[
            pltpu.VMEM((ch, KCH), i32),
            pltpu.VMEM((KCH, d), f32),
            pltpu.VMEM_SHARED((npad, d), f32),
        ],
    )
    def deg(c_hbm, ones_hbm, z_hbm, out_hbm, cidx_v, ones_v, acc_sh):
        c = lax.axis_index("c")
        s = lax.axis_index("s")
        w = c * NSUB + s
        pltpu.sync_copy(z_hbm.at[pl.ds(s * rows_per, rows_per)],
                        acc_sh.at[pl.ds(s * rows_per, rows_per)])
        pltpu.sync_copy(c_hbm.at[w], cidx_v)
        pltpu.sync_copy(ones_hbm, ones_v)
        plsc.subcore_barrier()
        for j in range(ch):
            pltpu.sync_copy(ones_v, acc_sh.at[cidx_v.at[j]], add=True)
        plsc.subcore_barrier()
        pltpu.sync_copy(acc_sh.at[pl.ds(s * rows_per, rows_per)],
                        out_hbm.at[pl.ds(c * npad + s * rows_per, rows_per)])

    return deg(cidx, jnp.ones((KCH, d), f32), jnp.zeros((npad, d), f32))


def _pre_body(x_ref, w_ref, degp_ref, y_ref):
    dis = _dis_of(degp_ref)
    y_ref[...] = jnp.dot(x_ref[...], w_ref[...],
                         preferred_element_type=f32) * dis


def _gcn_pre(x, w, degp):
    n, din = x.shape
    hid = w.shape[1]
    return pl.pallas_call(
        _pre_body,
        in_specs=[
            pl.BlockSpec((n, din), lambda: (0, 0)),
            pl.BlockSpec((din, hid), lambda: (0, 0)),
            pl.BlockSpec((2, n, 1), lambda: (0, 0, 0)),
        ],
        out_specs=pl.BlockSpec((n, hid), lambda: (0, 0)),
        out_shape=jax.ShapeDtypeStruct((n, hid), f32),
    )(x, w, degp)


def _bn_tanh(s1, dis, b_ref, g_ref, be_ref):
    h = jnp.tanh(s1 * dis + b_ref[...])
    m = jnp.mean(h, axis=0, keepdims=True)
    v = jnp.mean((h - m) ** 2, axis=0, keepdims=True)
    return (h - m) * lax.rsqrt(v + 1e-5) * g_ref[...] + be_ref[...]


def _mid_body(parts_ref, degp_ref, b_ref, g_ref, be_ref, w2_ref, y2_ref):
    dis = _dis_of(degp_ref)
    t1 = _bn_tanh(parts_ref[0] + parts_ref[1], dis, b_ref, g_ref, be_ref)
    y2_ref[...] = jnp.dot(t1, w2_ref[...], preferred_element_type=f32) * dis


def _gcn_mid(parts, degp, b, g, be, w2):
    _, n, hid = parts.shape
    h2 = w2.shape[1]
    return pl.pallas_call(
        _mid_body,
        in_specs=[
            pl.BlockSpec((2, n, hid), lambda: (0, 0, 0)),
            pl.BlockSpec((2, n, 1), lambda: (0, 0, 0)),
            pl.BlockSpec((1, hid), lambda: (0, 0)),
            pl.BlockSpec((1, hid), lambda: (0, 0)),
            pl.BlockSpec((1, hid), lambda: (0, 0)),
            pl.BlockSpec((hid, h2), lambda: (0, 0)),
        ],
        out_specs=pl.BlockSpec((n, h2), lambda: (0, 0)),
        out_shape=jax.ShapeDtypeStruct((n, h2), f32),
    )(parts, degp, b, g, be, w2)


def _post_body(parts_ref, degp_ref, b_ref, g_ref, be_ref, h_ref):
    dis = _dis_of(degp_ref)
    h_ref[...] = _bn_tanh(parts_ref[0] + parts_ref[1], dis, b_ref, g_ref,
                          be_ref)


def _gcn_post(parts, degp, b, g, be):
    _, n, hid = parts.shape
    return pl.pallas_call(
        _post_body,
        in_specs=[
            pl.BlockSpec((2, n, hid), lambda: (0, 0, 0)),
            pl.BlockSpec((2, n, 1), lambda: (0, 0, 0)),
            pl.BlockSpec((1, hid), lambda: (0, 0)),
            pl.BlockSpec((1, hid), lambda: (0, 0)),
            pl.BlockSpec((1, hid), lambda: (0, 0)),
        ],
        out_specs=pl.BlockSpec((n, hid), lambda: (0, 0)),
        out_shape=jax.ShapeDtypeStruct((n, hid), f32),
    )(parts, degp, b, g, be)


def _gat_body(hag_f, hag_t, hab_t, bag_col, bab_col_t, bab_row_t, w_ref,
              as_ref, ad_ref, bias_ref, oag_ref, oab_ref,
              xlag_ref, asn_ref, m_ref, eye_ref, *, heads, c, concat, jt):
    i = pl.program_id(0)
    n = hag_f.shape[0]
    kt = hab_t.shape[0]

    @pl.when(i == 0)
    def _():
        xlag = jnp.dot(hag_f[...], w_ref[...], preferred_element_type=f32)
        xlag_ref[...] = xlag
        iotab = lax.broadcasted_iota(i32, (n, NBATCH), 1)
        onehot_bag = bag_col[...] == iotab
        ii = lax.broadcasted_iota(i32, (kt, kt), 0)
        jj = lax.broadcasted_iota(i32, (kt, kt), 1)
        eye_ref[...] = (ii == jj).astype(f32)
        for h in range(heads):
            xl_h = xlag[:, h * c:(h + 1) * c]
            asn_h = jnp.sum(xl_h * as_ref[h:h + 1, :], axis=1, keepdims=True)
            asn_ref[:, h:h + 1] = asn_h
            m_ref[h:h + 1, :] = jnp.max(
                jnp.where(onehot_bag, asn_h, _NEG), axis=0, keepdims=True)
        asn_ref[:, heads:heads + 1] = bag_col[...].astype(f32)

    eye = eye_ref[...]

    def tr(colv):
        return lax.dot_general(colv, eye, (((0,), (0,)), ((), ())),
                               preferred_element_type=f32)

    def trr(rowv):
        return lax.dot_general(eye, rowv, (((1,), (1,)), ((), ())),
                               preferred_element_type=f32)

    xlag_t = jnp.dot(hag_t[...], w_ref[...], preferred_element_type=f32)
    xlab_t = jnp.dot(hab_t[...], w_ref[...], preferred_element_type=f32)
    iotab_t = lax.broadcasted_iota(i32, (kt, NBATCH), 1)
    onehot_bab = bab_col_t[...] == iotab_t

    blo = bab_col_t[0, 0]
    bhi = bab_col_t[kt - 1, 0]
    bag_all = bag_col[...]
    jlo = jnp.sum((bag_all < blo).astype(i32))
    jhi = jnp.sum((bag_all <= bhi).astype(i32))
    t0 = jlo // jt
    t1 = (jhi + jt - 1) // jt

    adn_cols, aself_cols, adn_rows, amax_rows = [], [], [], []
    for h in range(heads):
        xlab_h = xlab_t[:, h * c:(h + 1) * c]
        adn_col = jnp.sum(xlab_h * ad_ref[h:h + 1, :], axis=1, keepdims=True)
        asnab_col = jnp.sum(xlab_h * as_ref[h:h + 1, :], axis=1, keepdims=True)
        mg_col = jnp.max(jnp.where(onehot_bab, m_ref[h:h + 1, :], _NEG),
                         axis=1, keepdims=True)
        al_self = _lrelu(asnab_col + adn_col)
        amax_col = jnp.maximum(_lrelu(mg_col + adn_col), al_self)
        adn_cols.append(adn_col)
        aself_cols.append(jnp.exp(al_self - amax_col))
        adn_rows.append(tr(adn_col))
        amax_rows.append(tr(amax_col))

    bab_row_f = bab_row_t[...].astype(f32)

    def jbody(t, carry):
        vs, dens = carry
        sl = pl.ds(t * jt, jt)
        xlg_full = xlag_ref[sl, :]
        asn_full = asn_ref[sl, :]
        mask = asn_full[:, heads:heads + 1] == bab_row_f
        nvs, ndens = [], []
        for h in range(heads):
            xlg = xlg_full[:, h * c:(h + 1) * c]
            asn_sl = asn_full[:, h:h + 1]
            al = _lrelu(asn_sl + adn_rows[h])
            ae = jnp.where(mask, jnp.exp(al - amax_rows[h]), 0.0)
            ndens.append(dens[h] + jnp.sum(ae, axis=0, keepdims=True))
            nvs.append(vs[h] + lax.dot_general(
                ae, xlg, (((0,), (0,)), ((), ())),
                preferred_element_type=f32))
        return tuple(nvs), tuple(ndens)

    vs0 = tuple(jnp.zeros((kt, c), f32) for _ in range(heads))
    dens0 = tuple(jnp.zeros((1, kt), f32) for _ in range(heads))
    vs, dens = lax.fori_loop(t0, t1, jbody, (vs0, dens0))

    ag_parts = []
    ab_parts = []
    for h in range(heads):
        xlab_h = xlab_t[:, h * c:(h + 1) * c]
        den_col = trr(dens[h])
        num = vs[h] + aself_cols[h] * xlab_h
        ab_parts.append(num / (den_col + aself_cols[h] + 1e-16))
        ag_parts.append(xlag_t[:, h * c:(h + 1) * c])
    if concat:
        oag = jnp.concatenate(ag_parts, axis=1)
        oab = jnp.concatenate(ab_parts, axis=1)
    else:
        oag = sum(ag_parts) * (1.0 / heads)
        oab = sum(ab_parts) * (1.0 / heads)
    oag_ref[...] = jnp.tanh(oag + bias_ref[...])
    oab_ref[...] = jnp.tanh(oab + bias_ref[...])


def _gat(hag, hab, bag_col, bab_col, bab_row, w, a_s, a_d, bias, concat, kt,
         jt=512):
    n, din = hag.shape
    heads, c = a_s.shape
    out_d = heads * c if concat else c
    body = functools.partial(_gat_body, heads=heads, c=c, concat=concat, jt=jt)
    return pl.pallas_call(
        body,
        grid=(n // kt,),
        in_specs=[
            pl.BlockSpec((n, din), lambda i: (0, 0)),
            pl.BlockSpec((kt, din), lambda i: (i, 0)),
            pl.BlockSpec((kt, din), lambda i: (i, 0)),
            pl.BlockSpec((n, 1), lambda i: (0, 0)),
            pl.BlockSpec((kt, 1), lambda i: (i, 0)),
            pl.BlockSpec((1, kt), lambda i: (0, i)),
            pl.BlockSpec((din, heads * c), lambda i: (0, 0)),
            pl.BlockSpec((heads, c), lambda i: (0, 0)),
            pl.BlockSpec((heads, c), lambda i: (0, 0)),
            pl.BlockSpec((1, out_d), lambda i: (0, 0)),
        ],
        out_specs=[
            pl.BlockSpec((kt, out_d), lambda i: (i, 0)),
            pl.BlockSpec((kt, out_d), lambda i: (i, 0)),
        ],
        out_shape=[
            jax.ShapeDtypeStruct((n, out_d), f32),
            jax.ShapeDtypeStruct((n, out_d), f32),
        ],
        scratch_shapes=[
            pltpu.VMEM((n, heads * c), f32),
            pltpu.VMEM((n, heads + 4), f32),
            pltpu.VMEM((8, NBATCH), f32),
            pltpu.VMEM((kt, kt), f32),
        ],
    )(hag, hag, hab, bag_col, bab_col, bab_row, w, a_s, a_d, bias)


def _mlp_body(h1_ref, h2_ref, w1_ref, b1_ref, w2_ref, b2_ref, out_ref):
    cat = jnp.concatenate([h1_ref[...], h2_ref[...]], axis=1)
    t = jnp.tanh(jnp.dot(cat, w1_ref[...], preferred_element_type=f32)
                 + b1_ref[...])
    out_ref[...] = jnp.dot(t, w2_ref[...],
                           preferred_element_type=f32) + b2_ref[...]


def _mlp(h1, h2, w1, b1, w2, b2):
    n, d1 = h1.shape
    d2 = h2.shape[1]
    hid = w1.shape[1]
    return pl.pallas_call(
        _mlp_body,
        in_specs=[
            pl.BlockSpec((n, d1), lambda: (0, 0)),
            pl.BlockSpec((n, d2), lambda: (0, 0)),
            pl.BlockSpec((d1 + d2, hid), lambda: (0, 0)),
            pl.BlockSpec((1, hid), lambda: (0, 0)),
            pl.BlockSpec((hid, 1), lambda: (0, 0)),
            pl.BlockSpec((1, 1), lambda: (0, 0)),
        ],
        out_specs=pl.BlockSpec((n, 1), lambda: (0, 0)),
        out_shape=jax.ShapeDtypeStruct((n, 1), f32),
    )(h1, h2, w1, b1, w2, b2)


def _edges_sc(ei, n):
    npad = n + 8 * NSUB
    sl = jnp.arange(n, dtype=ei.dtype)
    row = jnp.concatenate([ei[0], sl])
    col = jnp.concatenate([ei[1], sl])
    e = row.shape[0]
    epad = -(-e // (NW * KCH)) * (NW * KCH)
    pad = epad - e
    fill = jnp.full((pad,), npad - 1, dtype=ei.dtype)
    row = jnp.concatenate([row, fill]).reshape(NW, -1, KCH)
    col = jnp.concatenate([col, fill]).reshape(NW, -1, KCH)
    return row, col, npad


def _split_parts(out2, n, npad, d):
    return jnp.stack([out2[0:n, :d], out2[npad:npad + n, :d]])


def _gcn_graph(x, w1, b1, g1, be1, w2, b2, g2, be2, ei):
    n = x.shape[0]
    hid = w1.shape[1]
    ridx, cidx, npad = _edges_sc(ei, n)
    degp = _split_parts(_sc_deg(cidx, npad), n, npad, 1)
    y1 = _gcn_pre(x, w1, degp)
    y1p = jnp.concatenate([y1, jnp.zeros((npad - n, hid), f32)])
    s1 = _split_parts(_sc_segsum(y1p, ridx, cidx, npad), n, npad, hid)
    y2 = _gcn_mid(s1, degp, b1, g1, be1, w2)
    y2p = jnp.concatenate([y2, jnp.zeros((npad - n, w2.shape[1]), f32)])
    s2 = _split_parts(_sc_segsum(y2p, ridx, cidx, npad), n, npad, w2.shape[1])
    return _gcn_post(s2, degp, b2, g2, be2)


def kernel(ag_x, ag_edge_index, ag_x_batch, ab_x, ab_edge_index, ab_x_batch,
           W1_ag, b1_ag, W2_ag, b2_ag, W1_ab, b1_ab, W2_ab, b2_ab,
           g1_ag, be1_ag, g2_ag, be2_ag, g1_ab, be1_ab, g2_ab, be2_ab,
           Wg1, as1, ad1, bg1, Wg2, as2, ad2, bg2,
           Wc1_ag, bc1_ag, Wc2_ag, bc2_ag, Wc1_ab, bc1_ab, Wc2_ab, bc2_ab,
           *, kt=256):
    n_ag = ag_x.shape[0]
    n_ab = ab_x.shape[0]
    r2 = lambda v: v.reshape(1, -1)

    ag_h1 = _gcn_graph(ag_x, W1_ag, r2(b1_ag), r2(g1_ag), r2(be1_ag),
                       W2_ag, r2(b2_ag), r2(g2_ag), r2(be2_ag), ag_edge_index)
    ab_h1 = _gcn_graph(ab_x, W1_ab, r2(b1_ab), r2(g1_ab), r2(be1_ab),
                       W2_ab, r2(b2_ab), r2(g2_ab), r2(be2_ab), ab_edge_index)

    bag_col = ag_x_batch.reshape(n_ag, 1)
    bab_col = ab_x_batch.reshape(n_ab, 1)
    bab_row = ab_x_batch.reshape(1, n_ab)
    ag_g1, ab_g1 = _gat(ag_h1, ab_h1, bag_col, bab_col, bab_row,
                        Wg1, as1, ad1, r2(bg1), True, kt)
    ag_h2, ab_h2 = _gat(ag_g1, ab_g1, bag_col, bab_col, bab_row,
                        Wg2, as2, ad2, r2(bg2), False, kt)

    ag_out = _mlp(ag_h1, ag_h2, Wc1_ag, r2(bc1_ag), Wc2_ag, r2(bc2_ag))
    ab_out = _mlp(ab_h1, ab_h2, Wc1_ab, r2(bc1_ab), Wc2_ab, r2(bc2_ab))
    return (ag_out, ag_h1, ag_h2, ab_out, ab_h1, ab_h2)
